# Initial kernel scaffold; baseline (speedup 1.0000x reference)
import jax
import jax.numpy as jnp
from jax.experimental import pallas as pl


def kernel(inputs, gate_w, expert_ws):
    raise NotImplementedError("write your pallas kernel here")


# speedup vs baseline: 1.4760x; 1.4760x over previous
import functools

import jax
import jax.numpy as jnp
from jax.experimental import pallas as pl
from jax.experimental.pallas import tpu as pltpu

E = 8
K = 2
T = 2048
D = 1024
DFF = 2816

BLK = 256
NB = (T * K) // BLK + E
P = NB * BLK
NSPLIT = 2
DFF_C = DFF // NSPLIT

_INTERPRET = False


def _gate_body(x_ref, gw_ref, e1_ref, e2_ref, w1_ref, w2_ref):
    x = x_ref[...]
    gl = jax.lax.dot_general(x, gw_ref[...], (((1,), (1,)), ((), ())))
    iota = jax.lax.broadcasted_iota(jnp.int32, gl.shape, 1)
    m1 = jnp.max(gl, axis=1, keepdims=True)
    a1 = jnp.min(jnp.where(gl == m1, iota, E), axis=1, keepdims=True)
    masked = jnp.where(iota == a1, -jnp.inf, gl)
    m2 = jnp.max(masked, axis=1, keepdims=True)
    a2 = jnp.min(jnp.where(masked == m2, iota, E), axis=1, keepdims=True)
    p1 = 1.0 / (1.0 + jnp.exp(m2 - m1))
    e1_ref[...] = a1[:, 0]
    e2_ref[...] = a2[:, 0]
    w1_ref[...] = p1[:, 0]
    w2_ref[...] = 1.0 - p1[:, 0]


def _gate(inputs, gate_w):
    nblk = T // BLK
    return pl.pallas_call(
        _gate_body,
        grid=(nblk,),
        in_specs=[
            pl.BlockSpec((BLK, D), lambda i: (i, 0)),
            pl.BlockSpec((E, D), lambda i: (0, 0)),
        ],
        out_specs=[
            pl.BlockSpec((BLK,), lambda i: (i,)),
            pl.BlockSpec((BLK,), lambda i: (i,)),
            pl.BlockSpec((BLK,), lambda i: (i,)),
            pl.BlockSpec((BLK,), lambda i: (i,)),
        ],
        out_shape=[
            jax.ShapeDtypeStruct((T,), jnp.int32),
            jax.ShapeDtypeStruct((T,), jnp.int32),
            jax.ShapeDtypeStruct((T,), jnp.float32),
            jax.ShapeDtypeStruct((T,), jnp.float32),
        ],
        interpret=_INTERPRET,
    )(inputs, gate_w)


def _route(e1, e2, wa, wb):
    e_all = jnp.concatenate([e1, e2])
    w_all = jnp.concatenate([wa, wb])
    t_all = jnp.concatenate([jnp.arange(T, dtype=jnp.int32)] * 2)
    onehot = (e_all[:, None] == jnp.arange(E)[None, :]).astype(jnp.int32)
    cnt = jnp.sum(onehot, axis=0)
    blocks = (cnt + BLK - 1) // BLK
    cblocks = jnp.cumsum(blocks)
    nb = cblocks[-1]
    start_blk = jnp.concatenate([jnp.zeros((1,), jnp.int32), cblocks[:-1]])
    rank = jnp.cumsum(onehot, axis=0) - onehot
    pos = start_blk[e_all] * BLK + jnp.take_along_axis(rank, e_all[:, None], 1)[:, 0]
    sorted_token = jnp.zeros((P,), jnp.int32).at[pos].set(t_all)
    sorted_w = jnp.zeros((P,), jnp.float32).at[pos].set(w_all)
    be = jnp.searchsorted(cblocks, jnp.arange(NB, dtype=jnp.int32), side="right")
    be = jnp.clip(be, 0, E - 1).astype(jnp.int32)
    be = jnp.where(jnp.arange(NB) < nb, be, be[jnp.maximum(nb - 1, 0)])
    sinfo = jnp.concatenate([be, nb[None].astype(jnp.int32)])
    return sorted_token, sorted_w, pos, sinfo


def _gather_rows(inputs, sorted_token):
    return inputs[sorted_token]


def _ffn_body(s_ref, xs_ref, w_ref, wt_ref, y_ref):
    i = pl.program_id(0)
    j = pl.program_id(1)
    nb = s_ref[NB]

    @pl.when(i < nb)
    def _():
        x = xs_ref[...]
        w0 = w_ref[0, 0]
        w1 = w_ref[0, 1]
        w2 = w_ref[0, 2]
        a = jax.lax.dot_general(x, w0, (((1,), (1,)), ((), ())))
        b = jax.lax.dot_general(x, w2, (((1,), (1,)), ((), ())))
        h = a * jax.lax.logistic(a) * b
        part = jnp.dot(h, w1)

        @pl.when(j == 0)
        def _():
            y_ref[...] = part

        @pl.when(j > 0)
        def _():
            y_ref[...] = y_ref[...] + part

        @pl.when(j == NSPLIT - 1)
        def _():
            y_ref[...] = y_ref[...] * wt_ref[0, 0, :][:, None]


def _ffn(sinfo, xs, expert_ws, sorted_w):
    wt3 = sorted_w.reshape(NB, 1, BLK)
    grid_spec = pltpu.PrefetchScalarGridSpec(
        num_scalar_prefetch=1,
        grid=(NB, NSPLIT),
        in_specs=[
            pl.BlockSpec((BLK, D), lambda i, j, s: (i, 0)),
            pl.BlockSpec((1, 3, DFF_C, D),
                         lambda i, j, s: (jnp.clip(s[jnp.minimum(i, s[NB] - 1)], 0, E - 1), 0, j, 0)),
            pl.BlockSpec((1, 1, BLK), lambda i, j, s: (i, 0, 0)),
        ],
        out_specs=pl.BlockSpec((BLK, D), lambda i, j, s: (i, 0)),
    )
    return pl.pallas_call(
        _ffn_body,
        grid_spec=grid_spec,
        out_shape=jax.ShapeDtypeStruct((P, D), jnp.float32),
        compiler_params=pltpu.CompilerParams(
            dimension_semantics=("arbitrary", "arbitrary"),
        ),
        interpret=_INTERPRET,
    )(sinfo, xs, expert_ws, wt3)


def _combine(yw, pos):
    return yw[pos[:T]] + yw[pos[T:]]


def kernel(inputs, gate_w, expert_ws):
    e1, e2, wa, wb = _gate(inputs, gate_w)
    sorted_token, sorted_w, pos, sinfo = _route(e1, e2, wa, wb)
    xs = _gather_rows(inputs, sorted_token)
    yw = _ffn(sinfo, xs, expert_ws, sorted_w)
    return _combine(yw, pos)


# speedup vs baseline: 1.4918x; 1.0107x over previous
import functools

import jax
import jax.numpy as jnp
from jax.experimental import pallas as pl
from jax.experimental.pallas import tpu as pltpu

E = 8
K = 2
T = 2048
D = 1024
DFF = 2816

BLK = 256
NB = (T * K) // BLK + E
P = NB * BLK
NSPLIT = 2
DFF_C = DFF // NSPLIT

_INTERPRET = False


def _gate_body(x_ref, gw_ref, e1_ref, e2_ref, w1_ref, w2_ref):
    x = x_ref[...]
    gl = jax.lax.dot_general(x, gw_ref[...], (((1,), (1,)), ((), ())))
    iota = jax.lax.broadcasted_iota(jnp.int32, gl.shape, 1)
    m1 = jnp.max(gl, axis=1, keepdims=True)
    a1 = jnp.min(jnp.where(gl == m1, iota, E), axis=1, keepdims=True)
    masked = jnp.where(iota == a1, -jnp.inf, gl)
    m2 = jnp.max(masked, axis=1, keepdims=True)
    a2 = jnp.min(jnp.where(masked == m2, iota, E), axis=1, keepdims=True)
    p1 = 1.0 / (1.0 + jnp.exp(m2 - m1))
    e1_ref[...] = a1[:, 0]
    e2_ref[...] = a2[:, 0]
    w1_ref[...] = p1[:, 0]
    w2_ref[...] = 1.0 - p1[:, 0]


def _gate(inputs, gate_w):
    nblk = T // BLK
    return pl.pallas_call(
        _gate_body,
        grid=(nblk,),
        in_specs=[
            pl.BlockSpec((BLK, D), lambda i: (i, 0)),
            pl.BlockSpec((E, D), lambda i: (0, 0)),
        ],
        out_specs=[
            pl.BlockSpec((BLK,), lambda i: (i,)),
            pl.BlockSpec((BLK,), lambda i: (i,)),
            pl.BlockSpec((BLK,), lambda i: (i,)),
            pl.BlockSpec((BLK,), lambda i: (i,)),
        ],
        out_shape=[
            jax.ShapeDtypeStruct((T,), jnp.int32),
            jax.ShapeDtypeStruct((T,), jnp.int32),
            jax.ShapeDtypeStruct((T,), jnp.float32),
            jax.ShapeDtypeStruct((T,), jnp.float32),
        ],
        interpret=_INTERPRET,
    )(inputs, gate_w)


def _route(e1, e2, wa, wb):
    e_all = jnp.concatenate([e1, e2])
    w_all = jnp.concatenate([wa, wb])
    t_all = jnp.concatenate([jnp.arange(T, dtype=jnp.int32)] * 2)
    onehot = (e_all[:, None] == jnp.arange(E)[None, :]).astype(jnp.int32)
    cnt = jnp.sum(onehot, axis=0)
    blocks = (cnt + BLK - 1) // BLK
    cblocks = jnp.cumsum(blocks)
    nb = cblocks[-1]
    start_blk = jnp.concatenate([jnp.zeros((1,), jnp.int32), cblocks[:-1]])
    rank = jnp.cumsum(onehot, axis=0) - onehot
    pos = start_blk[e_all] * BLK + jnp.take_along_axis(rank, e_all[:, None], 1)[:, 0]
    sorted_token = jnp.zeros((P,), jnp.int32).at[pos].set(t_all)
    sorted_w = jnp.zeros((P,), jnp.float32).at[pos].set(w_all)
    be = jnp.searchsorted(cblocks, jnp.arange(NB, dtype=jnp.int32), side="right")
    be = jnp.clip(be, 0, E - 1).astype(jnp.int32)
    be = jnp.where(jnp.arange(NB) < nb, be, be[jnp.maximum(nb - 1, 0)])
    sinfo = jnp.concatenate([be, nb[None].astype(jnp.int32)])
    return sorted_token, sorted_w, pos, sinfo


def _gather_rows(inputs, sorted_token):
    return inputs[sorted_token]


def _ffn_body(s_ref, xs_ref, w_ref, wt_ref, y_ref):
    i = pl.program_id(0)
    j = pl.program_id(1)
    nb = s_ref[NB]

    @pl.when(i < nb)
    def _():
        x = xs_ref[...].astype(jnp.bfloat16)
        w0 = w_ref[0, 0].astype(jnp.bfloat16)
        w1 = w_ref[0, 1].astype(jnp.bfloat16)
        w2 = w_ref[0, 2].astype(jnp.bfloat16)
        a = jax.lax.dot_general(x, w0, (((1,), (1,)), ((), ())),
                                preferred_element_type=jnp.float32)
        b = jax.lax.dot_general(x, w2, (((1,), (1,)), ((), ())),
                                preferred_element_type=jnp.float32)
        h = a * jax.lax.logistic(a) * b
        part = jax.lax.dot_general(h.astype(jnp.bfloat16), w1,
                                   (((1,), (0,)), ((), ())),
                                   preferred_element_type=jnp.float32)

        @pl.when(j == 0)
        def _():
            y_ref[...] = part

        @pl.when(j > 0)
        def _():
            y_ref[...] = y_ref[...] + part

        @pl.when(j == NSPLIT - 1)
        def _():
            y_ref[...] = y_ref[...] * wt_ref[0, 0, :][:, None]


def _ffn(sinfo, xs, expert_ws, sorted_w):
    wt3 = sorted_w.reshape(NB, 1, BLK)
    grid_spec = pltpu.PrefetchScalarGridSpec(
        num_scalar_prefetch=1,
        grid=(NB, NSPLIT),
        in_specs=[
            pl.BlockSpec((BLK, D), lambda i, j, s: (i, 0)),
            pl.BlockSpec((1, 3, DFF_C, D),
                         lambda i, j, s: (jnp.clip(s[jnp.minimum(i, s[NB] - 1)], 0, E - 1), 0, j, 0)),
            pl.BlockSpec((1, 1, BLK), lambda i, j, s: (i, 0, 0)),
        ],
        out_specs=pl.BlockSpec((BLK, D), lambda i, j, s: (i, 0)),
    )
    return pl.pallas_call(
        _ffn_body,
        grid_spec=grid_spec,
        out_shape=jax.ShapeDtypeStruct((P, D), jnp.float32),
        compiler_params=pltpu.CompilerParams(
            dimension_semantics=("arbitrary", "arbitrary"),
        ),
        interpret=_INTERPRET,
    )(sinfo, xs, expert_ws, wt3)


def _combine(yw, pos):
    return yw[pos[:T]] + yw[pos[T:]]


def kernel(inputs, gate_w, expert_ws):
    e1, e2, wa, wb = _gate(inputs, gate_w)
    sorted_token, sorted_w, pos, sinfo = _route(e1, e2, wa, wb)
    xs = _gather_rows(inputs, sorted_token)
    yw = _ffn(sinfo, xs, expert_ws, sorted_w)
    return _combine(yw, pos)
